# baseline (speedup 1.0000x reference)
import jax
import jax.numpy as jnp
from jax.experimental import pallas as pl

_TOP_K = 2
_E = 8
_F = 768
_R = 16
_ER = _E * _R

_DN_RHS_T = (((1,), (1,)), ((), ()))


def _body(a_ref, bs_ref, p_ref, tok_ref, out_ref):
    tok = tok_ref[...]
    sim8 = jnp.dot(tok, p_ref[...], preferred_element_type=jnp.float32)
    row8 = jax.lax.broadcasted_iota(jnp.int32, (_E, _ER), 0)
    lane = jax.lax.broadcasted_iota(jnp.int32, (_E, _ER), 1)
    e8 = jnp.where(row8 == lane // _R, 1.0, 0.0)
    simw = jnp.abs(jnp.dot(sim8, e8, preferred_element_type=jnp.float32,
                           precision=jax.lax.Precision.HIGHEST))
    lanew = jax.lax.broadcasted_iota(jnp.int32, simw.shape, 1)
    gid = lanew // _R
    m1 = jnp.max(simw, axis=1, keepdims=True)
    g1 = jnp.min(jnp.where(simw == m1, lanew, _ER), axis=1, keepdims=True) // _R
    masked = jnp.where(gid == g1, -jnp.inf, simw)
    m2 = jnp.max(masked, axis=1, keepdims=True)
    g2 = jnp.min(jnp.where(masked == m2, lanew, _ER), axis=1, keepdims=True) // _R
    e2 = jnp.exp(m2 - m1)
    denom = 1.0 + e2
    cexp = jnp.where(gid == g1, 1.0 / denom, 0.0) + jnp.where(gid == g2, e2 / denom, 0.0)
    u = jnp.dot(tok, a_ref[...], preferred_element_type=jnp.float32)
    v = u * cexp
    out_ref[...] = jnp.dot(v, bs_ref[...], preferred_element_type=jnp.float32)


def kernel(x, lora_A, lora_B, prototypes, scaling):
    orig_shape = x.shape
    f_in = x.shape[-1]
    tok = x.reshape(-1, f_in)
    t = tok.shape[0]
    a2d = lora_A.reshape(_ER, _F).T
    scalf = jnp.asarray(scaling, jnp.float32)
    bs = (lora_B * scalf).transpose(0, 2, 1).reshape(_ER, _F)
    ptT = prototypes.T

    bt = 1024 if t % 1024 == 0 else t
    grid = (t // bt,)
    delta = pl.pallas_call(
        _body,
        grid=grid,
        in_specs=[
            pl.BlockSpec((_F, _ER), lambda i: (0, 0)),
            pl.BlockSpec((_ER, _F), lambda i: (0, 0)),
            pl.BlockSpec((_F, _E), lambda i: (0, 0)),
            pl.BlockSpec((bt, _F), lambda i: (i, 0)),
        ],
        out_specs=pl.BlockSpec((bt, _F), lambda i: (i, 0)),
        out_shape=jax.ShapeDtypeStruct((t, _F), jnp.float32),
    )(a2d, bs, ptT, tok)
    return delta.reshape(orig_shape[:-1] + (_F,))


# speedup vs baseline: 1.3937x; 1.3937x over previous
import jax
import jax.numpy as jnp
from jax.experimental import pallas as pl

_TOP_K = 2
_E = 8
_F = 768
_R = 16
_ER = _E * _R

_DN_RHS_T = (((1,), (1,)), ((), ()))


def _body(a_ref, bs_ref, p_ref, tok_ref, out_ref):
    tok = tok_ref[...]
    sim8 = jax.lax.dot_general(tok, p_ref[...], _DN_RHS_T,
                               preferred_element_type=jnp.float32)
    row8 = jax.lax.broadcasted_iota(jnp.int32, (_E, _ER), 0)
    lane = jax.lax.broadcasted_iota(jnp.int32, (_E, _ER), 1)
    e8 = jnp.where(row8 == lane // _R, 1.0, 0.0)
    simw = jnp.abs(jnp.dot(sim8, e8, preferred_element_type=jnp.float32,
                           precision=jax.lax.Precision.HIGHEST))
    gid = jax.lax.broadcasted_iota(jnp.int32, simw.shape, 1) // _R
    key = ((simw.view(jnp.int32) & ~jnp.int32(7))
           | (jnp.int32(_E - 1) - gid)).view(jnp.float32)
    m1 = jnp.max(key, axis=1, keepdims=True)
    masked = jnp.where(key == m1, -1.0, key)
    m2 = jnp.max(masked, axis=1, keepdims=True)
    e2 = jnp.exp(m2 - m1)
    denom = 1.0 + e2
    cexp = jnp.where(key == m1, 1.0 / denom, 0.0) + jnp.where(key == m2, e2 / denom, 0.0)
    u = jax.lax.dot_general(tok, a_ref[...], _DN_RHS_T,
                            preferred_element_type=jnp.float32)
    v = u * cexp
    out_ref[...] = jnp.dot(v, bs_ref[...], preferred_element_type=jnp.float32)


def kernel(x, lora_A, lora_B, prototypes, scaling):
    orig_shape = x.shape
    f_in = x.shape[-1]
    tok = x.reshape(-1, f_in)
    t = tok.shape[0]
    a2d = lora_A.reshape(_ER, _F)
    scalf = jnp.asarray(scaling, jnp.float32)
    bs = (lora_B * scalf).transpose(0, 2, 1).reshape(_ER, _F)

    bt = 1024 if t % 1024 == 0 else t
    grid = (t // bt,)
    delta = pl.pallas_call(
        _body,
        grid=grid,
        in_specs=[
            pl.BlockSpec((_ER, _F), lambda i: (0, 0)),
            pl.BlockSpec((_ER, _F), lambda i: (0, 0)),
            pl.BlockSpec((_E, _F), lambda i: (0, 0)),
            pl.BlockSpec((bt, _F), lambda i: (i, 0)),
        ],
        out_specs=pl.BlockSpec((bt, _F), lambda i: (i, 0)),
        out_shape=jax.ShapeDtypeStruct((t, _F), jnp.float32),
    )(a2d, bs, prototypes, tok)
    return delta.reshape(orig_shape[:-1] + (_F,))


# speedup vs baseline: 1.8739x; 1.3445x over previous
import jax
import jax.numpy as jnp
from jax.experimental import pallas as pl

_TOP_K = 2
_E = 8
_F = 768
_R = 16
_ER = _E * _R

_DN_RHS_T = (((1,), (1,)), ((), ()))


def _body(a_ref, bs_ref, p_ref, tok_ref, out_ref):
    tok = tok_ref[...]
    prows = jnp.concatenate(
        [jnp.broadcast_to(p_ref[e : e + 1, :], (_R, _F)) for e in range(_E)],
        axis=0,
    )
    simw = jnp.abs(jax.lax.dot_general(tok, prows, _DN_RHS_T,
                                       preferred_element_type=jnp.float32))
    m1 = jnp.max(simw, axis=1, keepdims=True)
    masked = jnp.where(simw == m1, -jnp.inf, simw)
    m2 = jnp.max(masked, axis=1, keepdims=True)
    e2 = jnp.exp(m2 - m1)
    denom = 1.0 + e2
    cexp = jnp.where(simw >= m2, jnp.exp(simw - m1), 0.0) / denom
    u = jax.lax.dot_general(tok, a_ref[...], _DN_RHS_T,
                            preferred_element_type=jnp.float32)
    v = u * cexp
    out_ref[...] = jnp.dot(v, bs_ref[...], preferred_element_type=jnp.float32)


def kernel(x, lora_A, lora_B, prototypes, scaling):
    orig_shape = x.shape
    f_in = x.shape[-1]
    tok = x.reshape(-1, f_in)
    t = tok.shape[0]
    a2d = lora_A.reshape(_ER, _F)
    scalf = jnp.asarray(scaling, jnp.float32)
    bs = (lora_B * scalf).transpose(0, 2, 1).reshape(_ER, _F)

    bt = 1024 if t % 1024 == 0 else t
    grid = (t // bt,)
    delta = pl.pallas_call(
        _body,
        grid=grid,
        in_specs=[
            pl.BlockSpec((_ER, _F), lambda i: (0, 0)),
            pl.BlockSpec((_ER, _F), lambda i: (0, 0)),
            pl.BlockSpec((_E, _F), lambda i: (0, 0)),
            pl.BlockSpec((bt, _F), lambda i: (i, 0)),
        ],
        out_specs=pl.BlockSpec((bt, _F), lambda i: (i, 0)),
        out_shape=jax.ShapeDtypeStruct((t, _F), jnp.float32),
    )(a2d, bs, prototypes, tok)
    return delta.reshape(orig_shape[:-1] + (_F,))
